# baseline (speedup 1.0000x reference)
import functools

import jax
import jax.numpy as jnp
from jax import lax
from jax.experimental import pallas as pl
from jax.experimental.pallas import tpu as pltpu
from jax.experimental.pallas import tpu_sc as plsc

N = 10000
E = 320000
F_IN = 128
HID = 128
NCLASS = 40
CPAD = 48

NW = 32
CHUNK = 128
SEG = 27
NSEG = 3
NBLK = SEG * NSEG
E_PAD = NW * CHUNK * NBLK
EPW = E_PAD // NW
RPT = 624
RPT_LAST = N - 15 * RPT

_mesh = plsc.VectorSubcoreMesh(core_axis_name="c", subcore_axis_name="s")


def _tile_rows_copy(s, src_at, dst_at):
    off = pl.multiple_of(s * RPT, 8)

    @pl.when(s < 15)
    def _():
        pltpu.sync_copy(src_at(off, RPT), dst_at(off, RPT))

    @pl.when(s == 15)
    def _():
        pltpu.sync_copy(src_at(15 * RPT, RPT_LAST), dst_at(15 * RPT, RPT_LAST))


def _make_deg_pass():
    @functools.partial(
        pl.kernel,
        mesh=_mesh,
        compiler_params=pltpu.CompilerParams(use_tc_tiling_on_sc=False),
        out_type=jax.ShapeDtypeStruct((2, N, 16), jnp.float32),
        scratch_types=[
            pltpu.VMEM((NSEG, SEG, CHUNK), jnp.int32),
            pltpu.VMEM((NSEG, SEG, CHUNK), jnp.float32),
            pltpu.VMEM((CHUNK, 16), jnp.float32),
            pltpu.VMEM_SHARED((N, 16), jnp.float32),
        ],
    )
    def deg_kernel(ones_hbm, col_hbm, w_hbm, out_hbm, col_v, w_v, msg_v, acc_sh):
        c = lax.axis_index("c")
        s = lax.axis_index("s")
        wid = c * 16 + s
        _tile_rows_copy(s,
                        lambda o, n: ones_hbm.at[pl.ds(o, n)],
                        lambda o, n: acc_sh.at[pl.ds(o, n)])
        pltpu.sync_copy(col_hbm.at[wid], col_v)
        pltpu.sync_copy(w_hbm.at[wid], w_v)
        plsc.subcore_barrier()

        for t in range(NSEG):
            def blk_body(b, _):
                def edge_body(k16, _):
                    w16 = w_v.at[t, b][pl.ds(k16 * 16, 16)]
                    for i in range(16):
                        mrow = msg_v.at[k16 * 16 + i]
                        mrow[:] = jnp.zeros((16,), jnp.float32) + w16[i]
                    return 0

                lax.fori_loop(0, CHUNK // 16, edge_body, 0)
                pltpu.sync_copy(msg_v, acc_sh.at[col_v.at[t, b]], add=True)
                return 0

            lax.fori_loop(0, SEG, blk_body, 0)
        plsc.subcore_barrier()
        _tile_rows_copy(s,
                        lambda o, n: acc_sh.at[pl.ds(o, n)],
                        lambda o, n: out_hbm.at[c, pl.ds(o, n)])

    return deg_kernel


def _make_edge_pass(D):
    @functools.partial(
        pl.kernel,
        mesh=_mesh,
        compiler_params=pltpu.CompilerParams(use_tc_tiling_on_sc=False),
        out_type=jax.ShapeDtypeStruct((2, N, D), jnp.float32),
        scratch_types=[
            pltpu.VMEM((SEG, CHUNK), jnp.int32),
            pltpu.VMEM((SEG, CHUNK), jnp.int32),
            pltpu.VMEM((SEG, CHUNK), jnp.float32),
            pltpu.VMEM((CHUNK, D), jnp.float32),
            pltpu.VMEM((CHUNK, D), jnp.float32),
            pltpu.VMEM_SHARED((N, D), jnp.float32),
            pltpu.SemaphoreType.DMA,
            pltpu.SemaphoreType.DMA,
        ],
    )
    def edge_kernel(hs_hbm, row_hbm, col_hbm, w_hbm, out_hbm,
                    row_v, col_v, w_v, rows_a, rows_b, acc_sh, sem_a, sem_b):
        c = lax.axis_index("c")
        s = lax.axis_index("s")
        wid = c * 16 + s
        _tile_rows_copy(s,
                        lambda o, n: hs_hbm.at[pl.ds(o, n)],
                        lambda o, n: acc_sh.at[pl.ds(o, n)])
        plsc.subcore_barrier()

        def gather(b, buf, sem):
            return pltpu.async_copy(hs_hbm.at[row_v.at[b]], buf, sem)

        def scale_scatter(b, buf):
            def edge_body(k16, _):
                w16 = w_v.at[b][pl.ds(k16 * 16, 16)]
                for i in range(16):
                    rr = buf.at[k16 * 16 + i]
                    wk = w16[i]
                    for j in range(D // 16):
                        sl = pl.ds(j * 16, 16)
                        rr[sl] = rr[sl] * wk
                return 0

            lax.fori_loop(0, CHUNK // 16, edge_body, 0)
            pltpu.sync_copy(buf, acc_sh.at[col_v.at[b]], add=True)

        def wait_gather(b, buf, sem):
            pltpu.make_async_copy(hs_hbm.at[row_v.at[b]], buf, sem).wait()

        for t in range(NSEG):
            pltpu.sync_copy(row_hbm.at[wid, t], row_v)
            pltpu.sync_copy(col_hbm.at[wid, t], col_v)
            pltpu.sync_copy(w_hbm.at[wid, t], w_v)
            gather(0, rows_a, sem_a)

            def blk_body(g, _):
                b0 = g * 2
                gather(b0 + 1, rows_b, sem_b)
                wait_gather(b0, rows_a, sem_a)
                scale_scatter(b0, rows_a)
                gather(b0 + 2, rows_a, sem_a)
                wait_gather(b0 + 1, rows_b, sem_b)
                scale_scatter(b0 + 1, rows_b)
                return 0

            lax.fori_loop(0, (SEG - 1) // 2, blk_body, 0)
            wait_gather(SEG - 1, rows_a, sem_a)
            scale_scatter(SEG - 1, rows_a)

        plsc.subcore_barrier()
        _tile_rows_copy(s,
                        lambda o, n: acc_sh.at[pl.ds(o, n)],
                        lambda o, n: out_hbm.at[c, pl.ds(o, n)])

    return edge_kernel


_deg_pass = _make_deg_pass()
_edge_pass_h = _make_edge_pass(HID)
_edge_pass_c = _make_edge_pass(CPAD)

_BLK = 2000
_GRID = N // _BLK


def _dinv_block(d0, d1):
    deg = d0[:, :1] + d1[:, :1] - 1.0
    return jnp.where(deg > 0, lax.rsqrt(deg), 0.0)


def _mm_scale_body(x_ref, w_ref, d0_ref, d1_ref, o_ref):
    dinv = _dinv_block(d0_ref[...], d1_ref[...])
    h = jnp.dot(x_ref[...], w_ref[...], preferred_element_type=jnp.float32)
    o_ref[...] = h * dinv


def _layer2_body(p0_ref, p1_ref, hs_ref, b1_ref, w2_ref, d0_ref, d1_ref, o_ref):
    dinv = _dinv_block(d0_ref[...], d1_ref[...])
    z = dinv * (p0_ref[...] + p1_ref[...] - hs_ref[...]) + b1_ref[...]
    z = jnp.maximum(z, 0.0)
    g = jnp.dot(z, w2_ref[...], preferred_element_type=jnp.float32)
    o_ref[...] = g * dinv


def _final_body(q0_ref, q1_ref, gs_ref, b2_ref, d0_ref, d1_ref, o_ref):
    dinv = _dinv_block(d0_ref[...], d1_ref[...])
    o = dinv * (q0_ref[...] + q1_ref[...] - gs_ref[...]) + b2_ref[...]
    mask = lax.broadcasted_iota(jnp.int32, (1, CPAD), 1) < NCLASS
    o = jnp.where(mask, o, -1e30)
    m = jnp.max(o, axis=1, keepdims=True)
    e = jnp.where(mask, jnp.exp(o - m), 0.0)
    lse = jnp.log(jnp.sum(e, axis=1, keepdims=True))
    o_ref[...] = o - m - lse


def _row_spec(d):
    return pl.BlockSpec((_BLK, d), lambda i: (i, 0))


def _full_spec(shape):
    return pl.BlockSpec(shape, lambda i: (0,) * len(shape))


def _mm_scale(x, W1, d0, d1):
    return pl.pallas_call(
        _mm_scale_body,
        grid=(_GRID,),
        in_specs=[_row_spec(F_IN), _full_spec((F_IN, HID)),
                  _row_spec(16), _row_spec(16)],
        out_specs=_row_spec(HID),
        out_shape=jax.ShapeDtypeStruct((N, HID), jnp.float32),
    )(x, W1, d0, d1)


def _layer2(p0, p1, hs, b1, W2p, d0, d1):
    return pl.pallas_call(
        _layer2_body,
        grid=(_GRID,),
        in_specs=[_row_spec(HID), _row_spec(HID), _row_spec(HID),
                  _full_spec((1, HID)), _full_spec((HID, CPAD)),
                  _row_spec(16), _row_spec(16)],
        out_specs=_row_spec(CPAD),
        out_shape=jax.ShapeDtypeStruct((N, CPAD), jnp.float32),
    )(p0, p1, hs, b1, W2p, d0, d1)


def _final(q0, q1, gs, b2p, d0, d1):
    return pl.pallas_call(
        _final_body,
        grid=(_GRID,),
        in_specs=[_row_spec(CPAD), _row_spec(CPAD), _row_spec(CPAD),
                  _full_spec((1, CPAD)), _row_spec(16), _row_spec(16)],
        out_specs=_row_spec(CPAD),
        out_shape=jax.ShapeDtypeStruct((N, CPAD), jnp.float32),
    )(q0, q1, gs, b2p, d0, d1)


def kernel(x, edge_index, edge_weight, W1, b1, W2, b2):
    row = edge_index[0]
    col = edge_index[1]
    pad = E_PAD - E
    shp = (NW, NSEG, SEG, CHUNK)
    rowp = jnp.concatenate([row, jnp.zeros((pad,), row.dtype)]).reshape(shp)
    colp = jnp.concatenate([col, jnp.zeros((pad,), col.dtype)]).reshape(shp)
    wp = jnp.concatenate([edge_weight, jnp.zeros((pad,), edge_weight.dtype)]).reshape(shp)

    ones16 = jnp.ones((N, 16), jnp.float32)
    degp = _deg_pass(ones16, colp, wp)
    d0 = degp[0]
    d1 = degp[1]

    hs = _mm_scale(x, W1, d0, d1)

    p = _edge_pass_h(hs, rowp, colp, wp)

    W2p = jnp.zeros((HID, CPAD), jnp.float32).at[:, :NCLASS].set(W2)
    b2p = jnp.zeros((1, CPAD), jnp.float32).at[0, :NCLASS].set(b2)
    gs = _layer2(p[0], p[1], hs, b1.reshape(1, HID), W2p, d0, d1)

    q = _edge_pass_c(gs, rowp, colp, wp)

    out = _final(q[0], q[1], gs, b2p, d0, d1)
    return out[:, :NCLASS]


# speedup vs baseline: 1.2863x; 1.2863x over previous
import functools

import jax
import jax.numpy as jnp
from jax import lax
from jax.experimental import pallas as pl
from jax.experimental.pallas import tpu as pltpu
from jax.experimental.pallas import tpu_sc as plsc

N = 10000
E = 320000
F_IN = 128
HID = 128
NCLASS = 40
D = 64

NW = 32
CHUNK = 128
SEG = 27
NSEG = 3
NBLK = SEG * NSEG
E_PAD = NW * CHUNK * NBLK
EPW = E_PAD // NW
RPT = 624
RPT_LAST = N - 15 * RPT

_mesh = plsc.VectorSubcoreMesh(core_axis_name="c", subcore_axis_name="s")


def _tile_rows_copy(s, src_at, dst_at):
    off = pl.multiple_of(s * RPT, 8)

    @pl.when(s < 15)
    def _():
        pltpu.sync_copy(src_at(off, RPT), dst_at(off, RPT))

    @pl.when(s == 15)
    def _():
        pltpu.sync_copy(src_at(15 * RPT, RPT_LAST), dst_at(15 * RPT, RPT_LAST))


def _make_deg_pass():
    @functools.partial(
        pl.kernel,
        mesh=_mesh,
        compiler_params=pltpu.CompilerParams(use_tc_tiling_on_sc=False),
        out_type=jax.ShapeDtypeStruct((2, N, 16), jnp.float32),
        scratch_types=[
            pltpu.VMEM((NSEG, SEG, CHUNK), jnp.int32),
            pltpu.VMEM((NSEG, SEG, CHUNK), jnp.float32),
            pltpu.VMEM((CHUNK, 16), jnp.float32),
            pltpu.VMEM_SHARED((N, 16), jnp.float32),
        ],
    )
    def deg_kernel(ones_hbm, col_hbm, w_hbm, out_hbm, col_v, w_v, msg_v, acc_sh):
        c = lax.axis_index("c")
        s = lax.axis_index("s")
        wid = c * 16 + s
        _tile_rows_copy(s,
                        lambda o, n: ones_hbm.at[pl.ds(o, n)],
                        lambda o, n: acc_sh.at[pl.ds(o, n)])
        pltpu.sync_copy(col_hbm.at[wid], col_v)
        pltpu.sync_copy(w_hbm.at[wid], w_v)
        plsc.subcore_barrier()

        for t in range(NSEG):
            def blk_body(b, _):
                def edge_body(k16, _):
                    w16 = w_v.at[t, b][pl.ds(k16 * 16, 16)]
                    for i in range(16):
                        mrow = msg_v.at[k16 * 16 + i]
                        mrow[:] = jnp.zeros((16,), jnp.float32) + w16[i]
                    return 0

                lax.fori_loop(0, CHUNK // 16, edge_body, 0)
                pltpu.sync_copy(msg_v, acc_sh.at[col_v.at[t, b]], add=True)
                return 0

            lax.fori_loop(0, SEG, blk_body, 0)
        plsc.subcore_barrier()
        _tile_rows_copy(s,
                        lambda o, n: acc_sh.at[pl.ds(o, n)],
                        lambda o, n: out_hbm.at[c, pl.ds(o, n)])

    return deg_kernel


def _make_edge_pass():
    @functools.partial(
        pl.kernel,
        mesh=_mesh,
        compiler_params=pltpu.CompilerParams(use_tc_tiling_on_sc=False),
        out_type=jax.ShapeDtypeStruct((2, N, D), jnp.float32),
        scratch_types=[
            pltpu.VMEM((SEG, CHUNK), jnp.int32),
            pltpu.VMEM((SEG, CHUNK), jnp.int32),
            pltpu.VMEM((SEG, CHUNK), jnp.float32),
            pltpu.VMEM((CHUNK, D), jnp.float32),
            pltpu.VMEM((CHUNK, D), jnp.float32),
            pltpu.VMEM_SHARED((N, D), jnp.float32),
            pltpu.VMEM_SHARED((N, D), jnp.float32),
            pltpu.SemaphoreType.DMA,
            pltpu.SemaphoreType.DMA,
        ],
    )
    def edge_kernel(feat_hbm, zeros_hbm, row_hbm, col_hbm, w_hbm, out_hbm,
                    row_v, col_v, w_v, rows_a, rows_b, feat_sh, acc_sh,
                    sem_a, sem_b):
        c = lax.axis_index("c")
        s = lax.axis_index("s")
        wid = c * 16 + s
        _tile_rows_copy(s,
                        lambda o, n: feat_hbm.at[pl.ds(o, n)],
                        lambda o, n: feat_sh.at[pl.ds(o, n)])
        @pl.when(c == 0)
        def _():
            _tile_rows_copy(s,
                            lambda o, n: feat_hbm.at[pl.ds(o, n)],
                            lambda o, n: acc_sh.at[pl.ds(o, n)])

        @pl.when(c == 1)
        def _():
            _tile_rows_copy(s,
                            lambda o, n: zeros_hbm.at[pl.ds(o, n)],
                            lambda o, n: acc_sh.at[pl.ds(o, n)])

        plsc.subcore_barrier()

        def gather(b, buf, sem):
            return pltpu.async_copy(feat_sh.at[row_v.at[b]], buf, sem)

        def scale_scatter(b, buf):
            def edge_body(k16, _):
                w16 = w_v.at[b][pl.ds(k16 * 16, 16)]
                for i in range(16):
                    rr = buf.at[k16 * 16 + i]
                    wk = w16[i]
                    for j in range(D // 16):
                        sl = pl.ds(j * 16, 16)
                        rr[sl] = rr[sl] * wk
                return 0

            lax.fori_loop(0, CHUNK // 16, edge_body, 0)
            pltpu.sync_copy(buf, acc_sh.at[col_v.at[b]], add=True)

        def wait_gather(b, buf, sem):
            pltpu.make_async_copy(feat_sh.at[row_v.at[b]], buf, sem).wait()

        for t in range(NSEG):
            pltpu.sync_copy(row_hbm.at[wid, t], row_v)
            pltpu.sync_copy(col_hbm.at[wid, t], col_v)
            pltpu.sync_copy(w_hbm.at[wid, t], w_v)
            gather(0, rows_a, sem_a)

            def blk_body(g, _):
                b0 = g * 2
                gather(b0 + 1, rows_b, sem_b)
                wait_gather(b0, rows_a, sem_a)
                scale_scatter(b0, rows_a)
                gather(b0 + 2, rows_a, sem_a)
                wait_gather(b0 + 1, rows_b, sem_b)
                scale_scatter(b0 + 1, rows_b)
                return 0

            lax.fori_loop(0, (SEG - 1) // 2, blk_body, 0)
            wait_gather(SEG - 1, rows_a, sem_a)
            scale_scatter(SEG - 1, rows_a)

        plsc.subcore_barrier()
        _tile_rows_copy(s,
                        lambda o, n: acc_sh.at[pl.ds(o, n)],
                        lambda o, n: out_hbm.at[c, pl.ds(o, n)])

    return edge_kernel


_deg_pass = _make_deg_pass()
_edge_pass = _make_edge_pass()

_BLK = 2000
_GRID = N // _BLK


def _dinv_block(d0, d1):
    deg = d0[:, :1] + d1[:, :1] - 1.0
    return jnp.where(deg > 0, lax.rsqrt(deg), 0.0)


def _mm_scale_body(x_ref, w_ref, d0_ref, d1_ref, lo_ref, hi_ref):
    dinv = _dinv_block(d0_ref[...], d1_ref[...])
    h = jnp.dot(x_ref[...], w_ref[...], preferred_element_type=jnp.float32)
    hs = h * dinv
    lo_ref[...] = hs[:, :D]
    hi_ref[...] = hs[:, D:]


def _layer2_body(pl0_ref, pl1_ref, ph0_ref, ph1_ref, b1_ref, w2_ref,
                 d0_ref, d1_ref, o_ref):
    dinv = _dinv_block(d0_ref[...], d1_ref[...])
    zs = jnp.concatenate([pl0_ref[...] + pl1_ref[...],
                          ph0_ref[...] + ph1_ref[...]], axis=1)
    z = jnp.maximum(dinv * zs + b1_ref[...], 0.0)
    g = jnp.dot(z, w2_ref[...], preferred_element_type=jnp.float32)
    o_ref[...] = g * dinv


def _final_body(q0_ref, q1_ref, b2_ref, d0_ref, d1_ref, o_ref):
    dinv = _dinv_block(d0_ref[...], d1_ref[...])
    o = dinv * (q0_ref[...] + q1_ref[...]) + b2_ref[...]
    mask = lax.broadcasted_iota(jnp.int32, (1, D), 1) < NCLASS
    o = jnp.where(mask, o, -1e30)
    m = jnp.max(o, axis=1, keepdims=True)
    e = jnp.where(mask, jnp.exp(o - m), 0.0)
    lse = jnp.log(jnp.sum(e, axis=1, keepdims=True))
    o_ref[...] = o - m - lse


def _row_spec(d):
    return pl.BlockSpec((_BLK, d), lambda i: (i, 0))


def _full_spec(shape):
    return pl.BlockSpec(shape, lambda i: (0,) * len(shape))


def _mm_scale(x, W1, d0, d1):
    return pl.pallas_call(
        _mm_scale_body,
        grid=(_GRID,),
        in_specs=[_row_spec(F_IN), _full_spec((F_IN, HID)),
                  _row_spec(16), _row_spec(16)],
        out_specs=[_row_spec(D), _row_spec(D)],
        out_shape=[jax.ShapeDtypeStruct((N, D), jnp.float32),
                   jax.ShapeDtypeStruct((N, D), jnp.float32)],
    )(x, W1, d0, d1)


def _layer2(pl0, pl1, ph0, ph1, b1, W2p, d0, d1):
    return pl.pallas_call(
        _layer2_body,
        grid=(_GRID,),
        in_specs=[_row_spec(D), _row_spec(D), _row_spec(D), _row_spec(D),
                  _full_spec((1, HID)), _full_spec((HID, D)),
                  _row_spec(16), _row_spec(16)],
        out_specs=_row_spec(D),
        out_shape=jax.ShapeDtypeStruct((N, D), jnp.float32),
    )(pl0, pl1, ph0, ph1, b1, W2p, d0, d1)


def _final(q0, q1, b2p, d0, d1):
    return pl.pallas_call(
        _final_body,
        grid=(_GRID,),
        in_specs=[_row_spec(D), _row_spec(D),
                  _full_spec((1, D)), _row_spec(16), _row_spec(16)],
        out_specs=_row_spec(D),
        out_shape=jax.ShapeDtypeStruct((N, D), jnp.float32),
    )(q0, q1, b2p, d0, d1)


def kernel(x, edge_index, edge_weight, W1, b1, W2, b2):
    row = edge_index[0]
    col = edge_index[1]
    pad = E_PAD - E
    shp = (NW, NSEG, SEG, CHUNK)
    rowp = jnp.concatenate([row, jnp.zeros((pad,), row.dtype)]).reshape(shp)
    colp = jnp.concatenate([col, jnp.zeros((pad,), col.dtype)]).reshape(shp)
    wp = jnp.concatenate([edge_weight, jnp.zeros((pad,), edge_weight.dtype)]).reshape(shp)

    ones16 = jnp.ones((N, 16), jnp.float32)
    degp = _deg_pass(ones16, colp, wp)
    d0 = degp[0]
    d1 = degp[1]

    hs_lo, hs_hi = _mm_scale(x, W1, d0, d1)

    zeros64 = jnp.zeros((N, D), jnp.float32)
    p_lo = _edge_pass(hs_lo, zeros64, rowp, colp, wp)
    p_hi = _edge_pass(hs_hi, zeros64, rowp, colp, wp)

    W2p = jnp.zeros((HID, D), jnp.float32).at[:, :NCLASS].set(W2)
    b2p = jnp.zeros((1, D), jnp.float32).at[0, :NCLASS].set(b2)
    gs = _layer2(p_lo[0], p_lo[1], p_hi[0], p_hi[1],
                 b1.reshape(1, HID), W2p, d0, d1)

    q = _edge_pass(gs, zeros64, rowp, colp, wp)

    out = _final(q[0], q[1], b2p, d0, d1)
    return out[:, :NCLASS]


# speedup vs baseline: 1.4260x; 1.1086x over previous
import functools

import jax
import jax.numpy as jnp
from jax import lax
from jax.experimental import pallas as pl
from jax.experimental.pallas import tpu as pltpu
from jax.experimental.pallas import tpu_sc as plsc

N = 10000
E = 320000
F_IN = 128
HID = 128
NCLASS = 40
D = 64

NW = 32
CHUNK = 128
SEG = 27
NSEG = 3
NBLK = SEG * NSEG
E_PAD = NW * CHUNK * NBLK
EPW = E_PAD // NW
RPT = 624
RPT_LAST = N - 15 * RPT

_mesh = plsc.VectorSubcoreMesh(core_axis_name="c", subcore_axis_name="s")


def _tile_rows_copy(s, src_at, dst_at):
    off = pl.multiple_of(s * RPT, 8)

    @pl.when(s < 15)
    def _():
        pltpu.sync_copy(src_at(off, RPT), dst_at(off, RPT))

    @pl.when(s == 15)
    def _():
        pltpu.sync_copy(src_at(15 * RPT, RPT_LAST), dst_at(15 * RPT, RPT_LAST))


def _make_deg_pass():
    @functools.partial(
        pl.kernel,
        mesh=_mesh,
        compiler_params=pltpu.CompilerParams(use_tc_tiling_on_sc=False),
        out_type=jax.ShapeDtypeStruct((2, N, 16), jnp.float32),
        scratch_types=[
            pltpu.VMEM((NSEG, SEG, CHUNK), jnp.int32),
            pltpu.VMEM((NSEG, SEG, CHUNK), jnp.float32),
            pltpu.VMEM((CHUNK, 16), jnp.float32),
            pltpu.VMEM_SHARED((N, 16), jnp.float32),
        ],
    )
    def deg_kernel(ones_hbm, col_hbm, w_hbm, out_hbm, col_v, w_v, msg_v, acc_sh):
        c = lax.axis_index("c")
        s = lax.axis_index("s")
        wid = c * 16 + s
        _tile_rows_copy(s,
                        lambda o, n: ones_hbm.at[pl.ds(o, n)],
                        lambda o, n: acc_sh.at[pl.ds(o, n)])
        pltpu.sync_copy(col_hbm.at[wid], col_v)
        pltpu.sync_copy(w_hbm.at[wid], w_v)
        plsc.subcore_barrier()

        for t in range(NSEG):
            def blk_body(b, _):
                def edge_body(k16, _):
                    w16 = w_v.at[t, b][pl.ds(k16 * 16, 16)]
                    for i in range(16):
                        mrow = msg_v.at[k16 * 16 + i]
                        mrow[:] = jnp.zeros((16,), jnp.float32) + w16[i]
                    return 0

                lax.fori_loop(0, CHUNK // 16, edge_body, 0)
                pltpu.sync_copy(msg_v, acc_sh.at[col_v.at[t, b]], add=True)
                return 0

            lax.fori_loop(0, SEG, blk_body, 0)
        plsc.subcore_barrier()
        _tile_rows_copy(s,
                        lambda o, n: acc_sh.at[pl.ds(o, n)],
                        lambda o, n: out_hbm.at[c, pl.ds(o, n)])

    return deg_kernel


def _make_edge_pass():
    scratch = [
        pltpu.VMEM((SEG, CHUNK), jnp.int32),
        pltpu.VMEM((SEG, CHUNK), jnp.int32),
        pltpu.VMEM((SEG, CHUNK), jnp.float32),
    ]
    scratch += [pltpu.VMEM((CHUNK, D), jnp.float32) for _ in range(3)]
    scratch += [pltpu.VMEM_SHARED((N, D), jnp.float32)] * 2
    scratch += [pltpu.SemaphoreType.DMA] * 6

    @functools.partial(
        pl.kernel,
        mesh=_mesh,
        compiler_params=pltpu.CompilerParams(use_tc_tiling_on_sc=False),
        out_type=jax.ShapeDtypeStruct((2, N, D), jnp.float32),
        scratch_types=scratch,
    )
    def edge_kernel(feat_hbm, zeros_hbm, row_hbm, col_hbm, w_hbm, out_hbm,
                    row_v, col_v, w_v, buf0, buf1, buf2, feat_sh, acc_sh,
                    g0, g1, g2, s0, s1, s2):
        c = lax.axis_index("c")
        s = lax.axis_index("s")
        wid = c * 16 + s
        _tile_rows_copy(s,
                        lambda o, n: feat_hbm.at[pl.ds(o, n)],
                        lambda o, n: feat_sh.at[pl.ds(o, n)])
        @pl.when(c == 0)
        def _():
            _tile_rows_copy(s,
                            lambda o, n: feat_hbm.at[pl.ds(o, n)],
                            lambda o, n: acc_sh.at[pl.ds(o, n)])

        @pl.when(c == 1)
        def _():
            _tile_rows_copy(s,
                            lambda o, n: zeros_hbm.at[pl.ds(o, n)],
                            lambda o, n: acc_sh.at[pl.ds(o, n)])

        plsc.subcore_barrier()

        bufs = (buf0, buf1, buf2)
        gsem = (g0, g1, g2)
        ssem = (s0, s1, s2)

        def gather(b, k):
            pltpu.async_copy(feat_sh.at[row_v.at[b]], bufs[k], gsem[k])

        def wait_gather(b, k):
            pltpu.make_async_copy(feat_sh.at[row_v.at[b]], bufs[k],
                                  gsem[k]).wait()

        def scatter(b, k):
            pltpu.async_copy(bufs[k], acc_sh.at[col_v.at[b]], ssem[k],
                             add=True)

        def wait_scatter(b, k):
            pltpu.make_async_copy(bufs[k], acc_sh.at[col_v.at[b]],
                                  ssem[k]).wait()

        def scale(b, k):
            buf = bufs[k]

            def edge_body(k16, _):
                w16 = w_v.at[b][pl.ds(k16 * 16, 16)]
                for i in range(16):
                    rr = buf.at[k16 * 16 + i]
                    wk = w16[i]
                    for j in range(D // 16):
                        sl = pl.ds(j * 16, 16)
                        rr[sl] = rr[sl] * wk
                return 0

            lax.fori_loop(0, CHUNK // 16, edge_body, 0)

        for t in range(NSEG):
            pltpu.sync_copy(row_hbm.at[wid, t], row_v)
            pltpu.sync_copy(col_hbm.at[wid, t], col_v)
            pltpu.sync_copy(w_hbm.at[wid, t], w_v)
            gather(0, 0)
            gather(1, 1)

            def triple_body(g, _):
                for k in range(3):
                    b = g * 3 + k
                    wait_gather(b, k)
                    scale(b, k)
                    if k == 0:
                        @pl.when(g > 0)
                        def _():
                            wait_scatter(g * 3 - 1, 2)
                    else:
                        wait_scatter(b - 1, k - 1)
                    scatter(b, k)
                    if k == 0:
                        gather(b + 2, 2)
                    else:
                        @pl.when(b + 3 < SEG + 1)
                        def _():
                            gather(b + 2, (k + 2) % 3)
                return 0

            lax.fori_loop(0, SEG // 3, triple_body, 0)
            wait_scatter(SEG - 1, 2)

        plsc.subcore_barrier()
        _tile_rows_copy(s,
                        lambda o, n: acc_sh.at[pl.ds(o, n)],
                        lambda o, n: out_hbm.at[c, pl.ds(o, n)])

    return edge_kernel


_deg_pass = _make_deg_pass()
_edge_pass = _make_edge_pass()

_BLK = 2000
_GRID = N // _BLK


def _dinv_block(d0, d1):
    deg = d0[:, :1] + d1[:, :1] - 1.0
    return jnp.where(deg > 0, lax.rsqrt(deg), 0.0)


def _mm_scale_body(x_ref, w_ref, d0_ref, d1_ref, lo_ref, hi_ref):
    dinv = _dinv_block(d0_ref[...], d1_ref[...])
    h = jnp.dot(x_ref[...], w_ref[...], preferred_element_type=jnp.float32)
    hs = h * dinv
    lo_ref[...] = hs[:, :D]
    hi_ref[...] = hs[:, D:]


def _layer2_body(pl0_ref, pl1_ref, ph0_ref, ph1_ref, b1_ref, w2_ref,
                 d0_ref, d1_ref, o_ref):
    dinv = _dinv_block(d0_ref[...], d1_ref[...])
    zs = jnp.concatenate([pl0_ref[...] + pl1_ref[...],
                          ph0_ref[...] + ph1_ref[...]], axis=1)
    z = jnp.maximum(dinv * zs + b1_ref[...], 0.0)
    g = jnp.dot(z, w2_ref[...], preferred_element_type=jnp.float32)
    o_ref[...] = g * dinv


def _final_body(q0_ref, q1_ref, b2_ref, d0_ref, d1_ref, o_ref):
    dinv = _dinv_block(d0_ref[...], d1_ref[...])
    o = dinv * (q0_ref[...] + q1_ref[...]) + b2_ref[...]
    mask = lax.broadcasted_iota(jnp.int32, (1, D), 1) < NCLASS
    o = jnp.where(mask, o, -1e30)
    m = jnp.max(o, axis=1, keepdims=True)
    e = jnp.where(mask, jnp.exp(o - m), 0.0)
    lse = jnp.log(jnp.sum(e, axis=1, keepdims=True))
    o_ref[...] = o - m - lse


def _row_spec(d):
    return pl.BlockSpec((_BLK, d), lambda i: (i, 0))


def _full_spec(shape):
    return pl.BlockSpec(shape, lambda i: (0,) * len(shape))


def _mm_scale(x, W1, d0, d1):
    return pl.pallas_call(
        _mm_scale_body,
        grid=(_GRID,),
        in_specs=[_row_spec(F_IN), _full_spec((F_IN, HID)),
                  _row_spec(16), _row_spec(16)],
        out_specs=[_row_spec(D), _row_spec(D)],
        out_shape=[jax.ShapeDtypeStruct((N, D), jnp.float32),
                   jax.ShapeDtypeStruct((N, D), jnp.float32)],
    )(x, W1, d0, d1)


def _layer2(pl0, pl1, ph0, ph1, b1, W2p, d0, d1):
    return pl.pallas_call(
        _layer2_body,
        grid=(_GRID,),
        in_specs=[_row_spec(D), _row_spec(D), _row_spec(D), _row_spec(D),
                  _full_spec((1, HID)), _full_spec((HID, D)),
                  _row_spec(16), _row_spec(16)],
        out_specs=_row_spec(D),
        out_shape=jax.ShapeDtypeStruct((N, D), jnp.float32),
    )(pl0, pl1, ph0, ph1, b1, W2p, d0, d1)


def _final(q0, q1, b2p, d0, d1):
    return pl.pallas_call(
        _final_body,
        grid=(_GRID,),
        in_specs=[_row_spec(D), _row_spec(D),
                  _full_spec((1, D)), _row_spec(16), _row_spec(16)],
        out_specs=_row_spec(D),
        out_shape=jax.ShapeDtypeStruct((N, D), jnp.float32),
    )(q0, q1, b2p, d0, d1)


def kernel(x, edge_index, edge_weight, W1, b1, W2, b2):
    row = edge_index[0]
    col = edge_index[1]
    pad = E_PAD - E
    shp = (NW, NSEG, SEG, CHUNK)
    rowp = jnp.concatenate([row, jnp.zeros((pad,), row.dtype)]).reshape(shp)
    colp = jnp.concatenate([col, jnp.zeros((pad,), col.dtype)]).reshape(shp)
    wp = jnp.concatenate([edge_weight, jnp.zeros((pad,), edge_weight.dtype)]).reshape(shp)

    ones16 = jnp.ones((N, 16), jnp.float32)
    degp = _deg_pass(ones16, colp, wp)
    d0 = degp[0]
    d1 = degp[1]

    hs_lo, hs_hi = _mm_scale(x, W1, d0, d1)

    zeros64 = jnp.zeros((N, D), jnp.float32)
    p_lo = _edge_pass(hs_lo, zeros64, rowp, colp, wp)
    p_hi = _edge_pass(hs_hi, zeros64, rowp, colp, wp)

    W2p = jnp.zeros((HID, D), jnp.float32).at[:, :NCLASS].set(W2)
    b2p = jnp.zeros((1, D), jnp.float32).at[0, :NCLASS].set(b2)
    gs = _layer2(p_lo[0], p_lo[1], p_hi[0], p_hi[1],
                 b1.reshape(1, HID), W2p, d0, d1)

    q = _edge_pass(gs, zeros64, rowp, colp, wp)

    out = _final(q[0], q[1], b2p, d0, d1)
    return out[:, :NCLASS]


# speedup vs baseline: 1.4276x; 1.0012x over previous
import functools

import jax
import jax.numpy as jnp
from jax import lax
from jax.experimental import pallas as pl
from jax.experimental.pallas import tpu as pltpu
from jax.experimental.pallas import tpu_sc as plsc

N = 10000
E = 320000
F_IN = 128
HID = 128
NCLASS = 40
D = 64

NW = 32
CHUNK = 128
SEG = 27
NSEG = 3
NBLK = SEG * NSEG
E_PAD = NW * CHUNK * NBLK
EPW = E_PAD // NW
RPT = 624
RPT_LAST = N - 15 * RPT

@functools.lru_cache(maxsize=None)
def _mesh():
    return plsc.VectorSubcoreMesh(core_axis_name="c", subcore_axis_name="s")


def _tile_rows_copy(s, src_at, dst_at):
    off = pl.multiple_of(s * RPT, 8)

    @pl.when(s < 15)
    def _():
        pltpu.sync_copy(src_at(off, RPT), dst_at(off, RPT))

    @pl.when(s == 15)
    def _():
        pltpu.sync_copy(src_at(15 * RPT, RPT_LAST), dst_at(15 * RPT, RPT_LAST))


def _make_deg_pass():
    @functools.partial(
        pl.kernel,
        mesh=_mesh(),
        compiler_params=pltpu.CompilerParams(use_tc_tiling_on_sc=False),
        out_type=jax.ShapeDtypeStruct((2, N, 16), jnp.float32),
        scratch_types=[
            pltpu.VMEM((NSEG, SEG, CHUNK), jnp.int32),
            pltpu.VMEM((NSEG, SEG, CHUNK), jnp.float32),
            pltpu.VMEM((CHUNK, 16), jnp.float32),
            pltpu.VMEM_SHARED((N, 16), jnp.float32),
        ],
    )
    def deg_kernel(ones_hbm, col_hbm, w_hbm, out_hbm, col_v, w_v, msg_v, acc_sh):
        c = lax.axis_index("c")
        s = lax.axis_index("s")
        wid = c * 16 + s
        _tile_rows_copy(s,
                        lambda o, n: ones_hbm.at[pl.ds(o, n)],
                        lambda o, n: acc_sh.at[pl.ds(o, n)])
        pltpu.sync_copy(col_hbm.at[wid], col_v)
        pltpu.sync_copy(w_hbm.at[wid], w_v)
        plsc.subcore_barrier()

        for t in range(NSEG):
            def blk_body(b, _):
                def edge_body(k16, _):
                    w16 = w_v.at[t, b][pl.ds(k16 * 16, 16)]
                    for i in range(16):
                        mrow = msg_v.at[k16 * 16 + i]
                        mrow[:] = jnp.zeros((16,), jnp.float32) + w16[i]
                    return 0

                lax.fori_loop(0, CHUNK // 16, edge_body, 0)
                pltpu.sync_copy(msg_v, acc_sh.at[col_v.at[t, b]], add=True)
                return 0

            lax.fori_loop(0, SEG, blk_body, 0)
        plsc.subcore_barrier()
        _tile_rows_copy(s,
                        lambda o, n: acc_sh.at[pl.ds(o, n)],
                        lambda o, n: out_hbm.at[c, pl.ds(o, n)])

    return deg_kernel


def _make_edge_pass():
    scratch = [
        pltpu.VMEM((SEG, CHUNK), jnp.int32),
        pltpu.VMEM((SEG, CHUNK), jnp.int32),
        pltpu.VMEM((SEG, CHUNK), jnp.float32),
    ]
    scratch += [pltpu.VMEM((CHUNK, D), jnp.float32) for _ in range(3)]
    scratch += [pltpu.VMEM_SHARED((N, D), jnp.float32)] * 2
    scratch += [pltpu.SemaphoreType.DMA] * 6

    @functools.partial(
        pl.kernel,
        mesh=_mesh(),
        compiler_params=pltpu.CompilerParams(use_tc_tiling_on_sc=False),
        out_type=jax.ShapeDtypeStruct((2, N, D), jnp.float32),
        scratch_types=scratch,
    )
    def edge_kernel(feat_hbm, zeros_hbm, row_hbm, col_hbm, w_hbm, out_hbm,
                    row_v, col_v, w_v, buf0, buf1, buf2, feat_sh, acc_sh,
                    g0, g1, g2, s0, s1, s2):
        c = lax.axis_index("c")
        s = lax.axis_index("s")
        wid = c * 16 + s
        _tile_rows_copy(s,
                        lambda o, n: feat_hbm.at[pl.ds(o, n)],
                        lambda o, n: feat_sh.at[pl.ds(o, n)])
        @pl.when(c == 0)
        def _():
            _tile_rows_copy(s,
                            lambda o, n: feat_hbm.at[pl.ds(o, n)],
                            lambda o, n: acc_sh.at[pl.ds(o, n)])

        @pl.when(c == 1)
        def _():
            _tile_rows_copy(s,
                            lambda o, n: zeros_hbm.at[pl.ds(o, n)],
                            lambda o, n: acc_sh.at[pl.ds(o, n)])

        plsc.subcore_barrier()

        bufs = (buf0, buf1, buf2)
        gsem = (g0, g1, g2)
        ssem = (s0, s1, s2)

        def gather(b, k):
            pltpu.async_copy(feat_sh.at[row_v.at[b]], bufs[k], gsem[k])

        def wait_gather(b, k):
            pltpu.make_async_copy(feat_sh.at[row_v.at[b]], bufs[k],
                                  gsem[k]).wait()

        def scatter(b, k):
            pltpu.async_copy(bufs[k], acc_sh.at[col_v.at[b]], ssem[k],
                             add=True)

        def wait_scatter(b, k):
            pltpu.make_async_copy(bufs[k], acc_sh.at[col_v.at[b]],
                                  ssem[k]).wait()

        def scale(b, k):
            buf = bufs[k]

            def edge_body(k16, _):
                w16 = w_v.at[b][pl.ds(k16 * 16, 16)]
                for i in range(16):
                    rr = buf.at[k16 * 16 + i]
                    wk = w16[i]
                    for j in range(D // 16):
                        sl = pl.ds(j * 16, 16)
                        rr[sl] = rr[sl] * wk
                return 0

            lax.fori_loop(0, CHUNK // 16, edge_body, 0)

        for t in range(NSEG):
            pltpu.sync_copy(row_hbm.at[wid, t], row_v)
            pltpu.sync_copy(col_hbm.at[wid, t], col_v)
            pltpu.sync_copy(w_hbm.at[wid, t], w_v)
            gather(0, 0)
            gather(1, 1)

            def triple_body(g, _):
                for k in range(3):
                    b = g * 3 + k
                    wait_gather(b, k)
                    scale(b, k)
                    if k == 0:
                        @pl.when(g > 0)
                        def _():
                            wait_scatter(g * 3 - 1, 2)
                    else:
                        wait_scatter(b - 1, k - 1)
                    scatter(b, k)
                    if k == 0:
                        gather(b + 2, 2)
                    else:
                        @pl.when(b + 3 < SEG + 1)
                        def _():
                            gather(b + 2, (k + 2) % 3)
                return 0

            lax.fori_loop(0, SEG // 3, triple_body, 0)
            wait_scatter(SEG - 1, 2)

        plsc.subcore_barrier()
        _tile_rows_copy(s,
                        lambda o, n: acc_sh.at[pl.ds(o, n)],
                        lambda o, n: out_hbm.at[c, pl.ds(o, n)])

    return edge_kernel


_make_deg_pass = functools.lru_cache(maxsize=None)(_make_deg_pass)
_make_edge_pass = functools.lru_cache(maxsize=None)(_make_edge_pass)


def _deg_pass(*args):
    return _make_deg_pass()(*args)


def _edge_pass(*args):
    return _make_edge_pass()(*args)

_BLK = 2000
_GRID = N // _BLK


def _dinv_block(d0, d1):
    deg = d0[:, :1] + d1[:, :1] - 1.0
    return jnp.where(deg > 0, lax.rsqrt(deg), 0.0)


def _mm_scale_body(x_ref, w_ref, d0_ref, d1_ref, lo_ref, hi_ref):
    dinv = _dinv_block(d0_ref[...], d1_ref[...])
    h = jnp.dot(x_ref[...], w_ref[...], preferred_element_type=jnp.float32)
    hs = h * dinv
    lo_ref[...] = hs[:, :D]
    hi_ref[...] = hs[:, D:]


def _layer2_body(pl0_ref, pl1_ref, ph0_ref, ph1_ref, b1_ref, w2_ref,
                 d0_ref, d1_ref, o_ref):
    dinv = _dinv_block(d0_ref[...], d1_ref[...])
    zs = jnp.concatenate([pl0_ref[...] + pl1_ref[...],
                          ph0_ref[...] + ph1_ref[...]], axis=1)
    z = jnp.maximum(dinv * zs + b1_ref[...], 0.0)
    g = jnp.dot(z, w2_ref[...], preferred_element_type=jnp.float32)
    o_ref[...] = g * dinv


def _final_body(q0_ref, q1_ref, b2_ref, d0_ref, d1_ref, o_ref):
    dinv = _dinv_block(d0_ref[...], d1_ref[...])
    o = dinv * (q0_ref[...] + q1_ref[...]) + b2_ref[...]
    mask = lax.broadcasted_iota(jnp.int32, (1, D), 1) < NCLASS
    o = jnp.where(mask, o, -1e30)
    m = jnp.max(o, axis=1, keepdims=True)
    e = jnp.where(mask, jnp.exp(o - m), 0.0)
    lse = jnp.log(jnp.sum(e, axis=1, keepdims=True))
    o_ref[...] = o - m - lse


def _row_spec(d):
    return pl.BlockSpec((_BLK, d), lambda i: (i, 0))


def _full_spec(shape):
    return pl.BlockSpec(shape, lambda i: (0,) * len(shape))


def _mm_scale(x, W1, d0, d1):
    return pl.pallas_call(
        _mm_scale_body,
        grid=(_GRID,),
        in_specs=[_row_spec(F_IN), _full_spec((F_IN, HID)),
                  _row_spec(16), _row_spec(16)],
        out_specs=[_row_spec(D), _row_spec(D)],
        out_shape=[jax.ShapeDtypeStruct((N, D), jnp.float32),
                   jax.ShapeDtypeStruct((N, D), jnp.float32)],
    )(x, W1, d0, d1)


def _layer2(pl0, pl1, ph0, ph1, b1, W2p, d0, d1):
    return pl.pallas_call(
        _layer2_body,
        grid=(_GRID,),
        in_specs=[_row_spec(D), _row_spec(D), _row_spec(D), _row_spec(D),
                  _full_spec((1, HID)), _full_spec((HID, D)),
                  _row_spec(16), _row_spec(16)],
        out_specs=_row_spec(D),
        out_shape=jax.ShapeDtypeStruct((N, D), jnp.float32),
    )(pl0, pl1, ph0, ph1, b1, W2p, d0, d1)


def _final(q0, q1, b2p, d0, d1):
    return pl.pallas_call(
        _final_body,
        grid=(_GRID,),
        in_specs=[_row_spec(D), _row_spec(D),
                  _full_spec((1, D)), _row_spec(16), _row_spec(16)],
        out_specs=_row_spec(D),
        out_shape=jax.ShapeDtypeStruct((N, D), jnp.float32),
    )(q0, q1, b2p, d0, d1)


def kernel(x, edge_index, edge_weight, W1, b1, W2, b2):
    row = edge_index[0]
    col = edge_index[1]
    pad = E_PAD - E
    shp = (NW, NSEG, SEG, CHUNK)
    rowp = jnp.concatenate([row, jnp.zeros((pad,), row.dtype)]).reshape(shp)
    colp = jnp.concatenate([col, jnp.zeros((pad,), col.dtype)]).reshape(shp)
    wp = jnp.concatenate([edge_weight, jnp.zeros((pad,), edge_weight.dtype)]).reshape(shp)

    ones16 = jnp.ones((N, 16), jnp.float32)
    degp = _deg_pass(ones16, colp, wp)
    d0 = degp[0]
    d1 = degp[1]

    hs_lo, hs_hi = _mm_scale(x, W1, d0, d1)

    zeros64 = jnp.zeros((N, D), jnp.float32)
    p_lo = _edge_pass(hs_lo, zeros64, rowp, colp, wp)
    p_hi = _edge_pass(hs_hi, zeros64, rowp, colp, wp)

    W2p = jnp.zeros((HID, D), jnp.float32).at[:, :NCLASS].set(W2)
    b2p = jnp.zeros((1, D), jnp.float32).at[0, :NCLASS].set(b2)
    gs = _layer2(p_lo[0], p_lo[1], p_hi[0], p_hi[1],
                 b1.reshape(1, HID), W2p, d0, d1)

    q = _edge_pass(gs, zeros64, rowp, colp, wp)

    out = _final(q[0], q[1], b2p, d0, d1)
    return out[:, :NCLASS]
